# Initial kernel scaffold; baseline (speedup 1.0000x reference)
import jax
import jax.numpy as jnp
from jax.experimental import pallas as pl


def kernel(mod_feat_seq, p_in, W0, b0, W1, b1, mod_id_seq):
    raise NotImplementedError("write your pallas kernel here")


# speedup vs baseline: 1.5847x; 1.5847x over previous
import jax
import jax.numpy as jnp
from jax.experimental import pallas as pl
from jax.experimental.pallas import tpu as pltpu

B, L, E, FEAT, D_IN, D_H, D_OUT = 16384, 8, 8, 32, 64, 128, 32


def _fused_kernel(feat_ref, p_ref, w0_ref, b0_ref, w1_ref, b1_ref, ids_ref,
                  out_ref):
    bb = feat_ref.shape[0]
    p = p_ref[...]
    feats = feat_ref[...]
    ids = ids_ref[...]
    w0 = w0_ref[...]
    b0 = b0_ref[...]
    w1 = w1_ref[...]
    b1 = b1_ref[...]

    eidx = jax.lax.broadcasted_iota(jnp.int32, (bb, E * D_H), 1) // D_H

    for n in range(L):
        x = jnp.concatenate([p, feats[:, n * FEAT:(n + 1) * FEAT]], axis=1)
        h = jnp.tanh(
            jnp.dot(x, w0, preferred_element_type=jnp.float32) + b0)
        idn = ids[:, n:n + 1]
        h = jnp.where(eidx == idn, h, 0.0)
        p = jnp.dot(h, w1, preferred_element_type=jnp.float32)
        bsel = jnp.zeros((bb, D_OUT), jnp.float32)
        for i in range(E):
            bsel = jnp.where(idn == i, b1[i], bsel)
        p = p + bsel
    out_ref[...] = jnp.maximum(p, 0.0)


def kernel(mod_feat_seq, p_in, W0, b0, W1, b1, mod_id_seq):
    w0cat = jnp.transpose(W0, (2, 0, 1)).reshape(D_IN, E * D_H)
    b0cat = b0.reshape(1, E * D_H)
    w1cat = jnp.transpose(W1, (0, 2, 1)).reshape(E * D_H, D_OUT)
    feats = mod_feat_seq.reshape(B, L * FEAT)
    ids = mod_id_seq.astype(jnp.int32)

    BB = 1024
    grid = (B // BB,)
    return pl.pallas_call(
        _fused_kernel,
        grid=grid,
        in_specs=[
            pl.BlockSpec((BB, L * FEAT), lambda b: (b, 0)),
            pl.BlockSpec((BB, D_OUT), lambda b: (b, 0)),
            pl.BlockSpec((D_IN, E * D_H), lambda b: (0, 0)),
            pl.BlockSpec((1, E * D_H), lambda b: (0, 0)),
            pl.BlockSpec((E * D_H, D_OUT), lambda b: (0, 0)),
            pl.BlockSpec((E, D_OUT), lambda b: (0, 0)),
            pl.BlockSpec((BB, L), lambda b: (b, 0)),
        ],
        out_specs=pl.BlockSpec((BB, D_OUT), lambda b: (b, 0)),
        out_shape=jax.ShapeDtypeStruct((B, D_OUT), jnp.float32),
        compiler_params=pltpu.CompilerParams(
            dimension_semantics=("parallel",)),
    )(feats, p_in, w0cat, b0cat, w1cat, b1, ids)


# speedup vs baseline: 1.5899x; 1.0033x over previous
import jax
import jax.numpy as jnp
from jax.experimental import pallas as pl
from jax.experimental.pallas import tpu as pltpu

B, L, E, FEAT, D_IN, D_H, D_OUT = 16384, 8, 8, 32, 64, 128, 32


def _fused_kernel(feat_ref, p_ref, w0_ref, b0_ref, w1_ref, b1_ref, ids_ref,
                  out_ref):
    bb = feat_ref.shape[0]
    p = p_ref[...]
    feats = feat_ref[...]
    ids = ids_ref[...]
    w0 = w0_ref[...]
    b0 = b0_ref[...]
    w1 = w1_ref[...]
    b1 = b1_ref[...]

    eidx = jax.lax.broadcasted_iota(jnp.int32, (bb, E * D_H), 1) // D_H

    for n in range(L):
        x = jnp.concatenate([p, feats[:, n * FEAT:(n + 1) * FEAT]], axis=1)
        h = jnp.tanh(
            jnp.dot(x.astype(jnp.bfloat16), w0,
                    preferred_element_type=jnp.float32) + b0)
        idn = ids[:, n:n + 1]
        h = jnp.where(eidx == idn, h, 0.0)
        p = jnp.dot(h.astype(jnp.bfloat16), w1,
                    preferred_element_type=jnp.float32)
        bsel = jnp.zeros((bb, D_OUT), jnp.float32)
        for i in range(E):
            bsel = jnp.where(idn == i, b1[i], bsel)
        p = p + bsel
    out_ref[...] = jnp.maximum(p, 0.0)


def kernel(mod_feat_seq, p_in, W0, b0, W1, b1, mod_id_seq):
    w0cat = jnp.transpose(W0, (2, 0, 1)).reshape(D_IN, E * D_H)
    w0cat = w0cat.astype(jnp.bfloat16)
    b0cat = b0.reshape(1, E * D_H)
    w1cat = jnp.transpose(W1, (0, 2, 1)).reshape(E * D_H, D_OUT)
    w1cat = w1cat.astype(jnp.bfloat16)
    feats = mod_feat_seq.reshape(B, L * FEAT)
    ids = mod_id_seq.astype(jnp.int32)

    BB = 1024
    grid = (B // BB,)
    return pl.pallas_call(
        _fused_kernel,
        grid=grid,
        in_specs=[
            pl.BlockSpec((BB, L * FEAT), lambda b: (b, 0)),
            pl.BlockSpec((BB, D_OUT), lambda b: (b, 0)),
            pl.BlockSpec((D_IN, E * D_H), lambda b: (0, 0)),
            pl.BlockSpec((1, E * D_H), lambda b: (0, 0)),
            pl.BlockSpec((E * D_H, D_OUT), lambda b: (0, 0)),
            pl.BlockSpec((E, D_OUT), lambda b: (0, 0)),
            pl.BlockSpec((BB, L), lambda b: (b, 0)),
        ],
        out_specs=pl.BlockSpec((BB, D_OUT), lambda b: (b, 0)),
        out_shape=jax.ShapeDtypeStruct((B, D_OUT), jnp.float32),
        compiler_params=pltpu.CompilerParams(
            dimension_semantics=("parallel",)),
    )(feats, p_in, w0cat, b0cat, w1cat, b1, ids)


# speedup vs baseline: 1.8360x; 1.1548x over previous
import jax
import jax.numpy as jnp
from jax.experimental import pallas as pl
from jax.experimental.pallas import tpu as pltpu

B, L, E, FEAT, D_IN, D_H, D_OUT = 16384, 8, 8, 32, 64, 128, 32


def _fused_kernel(feat_ref, p_ref, w0_ref, b0_ref, w1_ref, b1_ref, ids_ref,
                  out_ref):
    bb = feat_ref.shape[0]
    p = p_ref[...]
    feats = feat_ref[...]
    ids = ids_ref[...]
    w0 = w0_ref[...]
    b0 = b0_ref[...]
    w1 = w1_ref[...]
    b1 = b1_ref[...]

    for n in range(L):
        x = jnp.concatenate([p, feats[:, n * FEAT:(n + 1) * FEAT]], axis=1)
        pre = jnp.dot(x.astype(jnp.bfloat16), w0,
                      preferred_element_type=jnp.float32) + b0
        idn = ids[:, n:n + 1]
        psel = pre[:, 0:D_H]
        for i in range(1, E):
            psel = jnp.where(idn == i, pre[:, i * D_H:(i + 1) * D_H], psel)
        h = jnp.tanh(psel).astype(jnp.bfloat16)
        o = jnp.dot(h, w1[0:D_H], preferred_element_type=jnp.float32) + b1[0]
        for i in range(1, E):
            oi = jnp.dot(h, w1[i * D_H:(i + 1) * D_H],
                         preferred_element_type=jnp.float32) + b1[i]
            o = jnp.where(idn == i, oi, o)
        p = o
    out_ref[...] = jnp.maximum(p, 0.0)


def kernel(mod_feat_seq, p_in, W0, b0, W1, b1, mod_id_seq):
    w0cat = jnp.transpose(W0, (2, 0, 1)).reshape(D_IN, E * D_H)
    w0cat = w0cat.astype(jnp.bfloat16)
    b0cat = b0.reshape(1, E * D_H)
    w1cat = jnp.transpose(W1, (0, 2, 1)).reshape(E * D_H, D_OUT)
    w1cat = w1cat.astype(jnp.bfloat16)
    feats = mod_feat_seq.reshape(B, L * FEAT)
    ids = mod_id_seq.astype(jnp.int32)

    BB = 1024
    grid = (B // BB,)
    return pl.pallas_call(
        _fused_kernel,
        grid=grid,
        in_specs=[
            pl.BlockSpec((BB, L * FEAT), lambda b: (b, 0)),
            pl.BlockSpec((BB, D_OUT), lambda b: (b, 0)),
            pl.BlockSpec((D_IN, E * D_H), lambda b: (0, 0)),
            pl.BlockSpec((1, E * D_H), lambda b: (0, 0)),
            pl.BlockSpec((E * D_H, D_OUT), lambda b: (0, 0)),
            pl.BlockSpec((E, D_OUT), lambda b: (0, 0)),
            pl.BlockSpec((BB, L), lambda b: (b, 0)),
        ],
        out_specs=pl.BlockSpec((BB, D_OUT), lambda b: (b, 0)),
        out_shape=jax.ShapeDtypeStruct((B, D_OUT), jnp.float32),
        compiler_params=pltpu.CompilerParams(
            dimension_semantics=("parallel",)),
    )(feats, p_in, w0cat, b0cat, w1cat, b1, ids)
